# baseline (speedup 1.0000x reference)
import functools

import jax
import jax.numpy as jnp
from jax import lax
from jax.experimental import pallas as pl
from jax.experimental.pallas import tpu as pltpu
from jax.experimental.pallas import tpu_sc as plsc

N_USERS = 50000
N_ITEMS = 50000
NN = N_USERS + N_ITEMS
D = 16
K_LAYERS = 3
NE = 3200000
BATCH = 4096

NC = 2
NS = 16
NW = NC * NS
LANES = 16

SUB = 128
BLK = 384
NSUB = BLK // SUB
NB = 264
NE_PAD = NW * BLK * NB

NNP = 100352
ROWS_SC = NNP // NS
ROWS_W = NNP // NW
CCH = 784


def _al8(x):
    return pl.multiple_of(x, 8)


_MESH = plsc.VectorSubcoreMesh(core_axis_name="c", subcore_axis_name="s")
_PARAMS = pltpu.CompilerParams(use_tc_tiling_on_sc=False)


@functools.partial(
    pl.kernel,
    out_type=jax.ShapeDtypeStruct((NC * NNP, D), jnp.float32),
    mesh=_MESH,
    compiler_params=_PARAMS,
    scratch_types=[
        pltpu.VMEM_SHARED((NNP, D), jnp.float32),
        pltpu.VMEM((4, 3, BLK), jnp.int32),
        pltpu.VMEM((2, BLK, D), jnp.float32),
        pltpu.VMEM((2, BLK, D), jnp.float32),
        pltpu.SemaphoreType.DMA,
        pltpu.SemaphoreType.DMA,
        pltpu.SemaphoreType.DMA,
        pltpu.SemaphoreType.DMA,
        pltpu.SemaphoreType.DMA,
        pltpu.SemaphoreType.DMA,
        pltpu.SemaphoreType.DMA,
        pltpu.SemaphoreType.DMA,
    ],
)
def _layer(e_hbm, z_hbm, ed_hbm, out_hbm,
           acc, eidx, rows, msg,
           si0, si1, si2, si3, sg0, sg1, ss0, ss1):
    cid = lax.axis_index("c")
    sid = lax.axis_index("s")
    wid = cid * NS + sid
    si = (si0, si1, si2, si3)
    sg = (sg0, sg1)
    ss = (ss0, ss1)

    r0 = _al8(sid * ROWS_SC)
    pltpu.sync_copy(z_hbm.at[pl.ds(r0, ROWS_SC)], acc.at[pl.ds(r0, ROWS_SC)])
    plsc.subcore_barrier()

    def fire_idx(b, q):
        pltpu.async_copy(ed_hbm.at[wid * NB + b], eidx.at[q], si[q])

    def wait_idx(q):
        pltpu.make_async_copy(ed_hbm.at[0], eidx.at[q], si[q]).wait()

    def fire_gather(q, p):
        pltpu.async_copy(e_hbm.at[eidx.at[q, 0]], rows.at[p], sg[p])

    def wait_gather(q, p):
        pltpu.make_async_copy(e_hbm.at[eidx.at[q, 0]], rows.at[p],
                              sg[p]).wait()

    def fire_scatter(q, p):
        pltpu.async_copy(msg.at[p], acc.at[eidx.at[q, 1]], ss[p], add=True)

    def wait_scatter(q, p):
        pltpu.make_async_copy(msg.at[p], acc.at[eidx.at[q, 1]], ss[p]).wait()

    fire_idx(0, 0)
    fire_idx(1, 1)
    fire_idx(2, 2)
    wait_idx(0)
    fire_gather(0, 0)

    def quarter_iter(bb, r):
        b = bb * 4 + r
        q = r
        qn = (r + 1) % 4
        p = r % 2
        pn = 1 - p

        wait_gather(q, p)

        @pl.when(b + 1 < NB)
        def _():
            wait_idx(qn)
            fire_gather(qn, pn)

        @pl.when(b >= 2)
        def _():
            wait_scatter((r + 2) % 4, p)

        @pl.loop(0, BLK // LANES)
        def _scale(mi):
            base = pl.multiple_of(mi * LANES, LANES)
            wvec = lax.bitcast_convert_type(
                eidx[q, 2, pl.ds(base, LANES)], jnp.float32)
            for j in range(LANES):
                msg[p, base + j] = rows[p, base + j] * wvec[j]

        fire_scatter(q, p)

        @pl.when(b + 3 < NB)
        def _():
            fire_idx(b + 3, (r + 3) % 4)

    @pl.loop(0, NB // 4)
    def _bb(bb):
        quarter_iter(bb, 0)
        quarter_iter(bb, 1)
        quarter_iter(bb, 2)
        quarter_iter(bb, 3)

    wait_scatter(2, 0)
    wait_scatter(3, 1)

    plsc.subcore_barrier()
    pltpu.sync_copy(acc.at[pl.ds(r0, ROWS_SC)],
                    out_hbm.at[pl.ds(_al8(cid * NNP + r0), ROWS_SC)])


def _combine_body(last, a_hbm, s_hbm, *refs):
    if last:
        ef_hbm, b0, b1, bs, sem = refs
    else:
        e_hbm, sn_hbm, b0, b1, bs, sem = refs
    wid = lax.axis_index("c") * NS + lax.axis_index("s")

    @pl.loop(0, ROWS_W // CCH)
    def _chunk(k):
        r0 = _al8(wid * ROWS_W + k * CCH)
        d0 = pltpu.async_copy(a_hbm.at[pl.ds(r0, CCH)], b0, sem)
        d1 = pltpu.async_copy(a_hbm.at[pl.ds(_al8(NNP + r0), CCH)], b1, sem)
        d2 = pltpu.async_copy(s_hbm.at[pl.ds(r0, CCH)], bs, sem)
        d0.wait()
        d1.wait()
        d2.wait()

        @pl.loop(0, CCH, unroll=8)
        def _row(r):
            e = b0[r] + b1[r]
            if last:
                b0[r] = (bs[r] + e) * jnp.float32(1.0 / (K_LAYERS + 1))
            else:
                b0[r] = e
                bs[r] = bs[r] + e

        if last:
            pltpu.sync_copy(b0, ef_hbm.at[pl.ds(r0, CCH)])
        else:
            d3 = pltpu.async_copy(b0, e_hbm.at[pl.ds(r0, CCH)], sem)
            d4 = pltpu.async_copy(bs, sn_hbm.at[pl.ds(r0, CCH)], sem)
            d3.wait()
            d4.wait()


_COMBINE_SCRATCH = [
    pltpu.VMEM((CCH, D), jnp.float32),
    pltpu.VMEM((CCH, D), jnp.float32),
    pltpu.VMEM((CCH, D), jnp.float32),
    pltpu.SemaphoreType.DMA,
]

_combine_mid = functools.partial(
    pl.kernel,
    out_type=(jax.ShapeDtypeStruct((NNP, D), jnp.float32),
              jax.ShapeDtypeStruct((NNP, D), jnp.float32)),
    mesh=_MESH,
    compiler_params=_PARAMS,
    scratch_types=_COMBINE_SCRATCH,
)(functools.partial(_combine_body, False))

_combine_last = functools.partial(
    pl.kernel,
    out_type=jax.ShapeDtypeStruct((NNP, D), jnp.float32),
    mesh=_MESH,
    compiler_params=_PARAMS,
    scratch_types=_COMBINE_SCRATCH,
)(functools.partial(_combine_body, True))


N_IDX = 3 * BATCH
G_PER_W = N_IDX // NW


@functools.partial(
    pl.kernel,
    out_type=jax.ShapeDtypeStruct((N_IDX, D), jnp.float32),
    mesh=_MESH,
    compiler_params=_PARAMS,
    scratch_types=[
        pltpu.VMEM((G_PER_W,), jnp.int32),
        pltpu.VMEM((G_PER_W, D), jnp.float32),
        pltpu.SemaphoreType.DMA,
    ],
)
def _batch_gather(e_hbm, idx_hbm, out_hbm, iv, rbuf, sem):
    wid = lax.axis_index("c") * NS + lax.axis_index("s")
    pltpu.sync_copy(idx_hbm.at[pl.ds(_al8(wid * G_PER_W), G_PER_W)], iv)
    descs = [
        pltpu.async_copy(e_hbm.at[iv.at[pl.ds(j * SUB, SUB)]],
                         rbuf.at[pl.ds(j * SUB, SUB)], sem)
        for j in range(G_PER_W // SUB)
    ]
    for d in descs:
        d.wait()
    pltpu.sync_copy(rbuf, out_hbm.at[pl.ds(_al8(wid * G_PER_W), G_PER_W)])


def kernel(users, pos_items, neg_items, user_emb, item_emb,
           edge_src, edge_dst, edge_w):
    E0 = jnp.concatenate([user_emb, item_emb], axis=0)
    E0 = jnp.pad(E0, ((0, NNP - NN), (0, 0)))
    Z = jnp.zeros((NNP, D), jnp.float32)
    pad = NE_PAD - NE
    srcb = jnp.pad(edge_src, (0, pad)).reshape(-1, BLK)
    dstb = jnp.pad(edge_dst, (0, pad)).reshape(-1, BLK)
    wb = jnp.pad(edge_w, (0, pad)).view(jnp.int32).reshape(-1, BLK)
    edata = jnp.stack([srcb, dstb, wb], axis=1)

    E, S = E0, E0
    for k in range(K_LAYERS):
        A = _layer(E, Z, edata)
        if k < K_LAYERS - 1:
            E, S = _combine_mid(A, S)
        else:
            E_final = _combine_last(A, S)

    idx = jnp.concatenate([users, N_USERS + pos_items, N_USERS + neg_items])
    rows = _batch_gather(E_final, idx)
    return (rows[:BATCH], rows[BATCH:2 * BATCH], rows[2 * BATCH:])


# speedup vs baseline: 1.1242x; 1.1242x over previous
import functools

import jax
import jax.numpy as jnp
from jax import lax
from jax.experimental import pallas as pl
from jax.experimental.pallas import tpu as pltpu
from jax.experimental.pallas import tpu_sc as plsc

N_USERS = 50000
N_ITEMS = 50000
NN = N_USERS + N_ITEMS
D = 16
K_LAYERS = 3
NE = 3200000
BATCH = 4096

NC = 2
NS = 16
NW = NC * NS
LANES = 16

SUB = 128
BLK = 384
NSUB = BLK // SUB
NB = 264
NE_PAD = NW * BLK * NB

NNP = 100352
ROWS_SC = NNP // NS
ROWS_W = NNP // NW
CCH = 784


def _al8(x):
    return pl.multiple_of(x, 8)


_MESH = plsc.VectorSubcoreMesh(core_axis_name="c", subcore_axis_name="s")
_PARAMS = pltpu.CompilerParams(use_tc_tiling_on_sc=False)


@functools.partial(
    pl.kernel,
    out_type=jax.ShapeDtypeStruct((NC * NNP, D), jnp.float32),
    mesh=_MESH,
    compiler_params=_PARAMS,
    scratch_types=[
        pltpu.VMEM_SHARED((NNP, D), jnp.float32),
        pltpu.VMEM((4, 3, BLK), jnp.int32),
        pltpu.VMEM((4, BLK, D), jnp.float32),
        pltpu.SemaphoreType.DMA,
        pltpu.SemaphoreType.DMA,
        pltpu.SemaphoreType.DMA,
        pltpu.SemaphoreType.DMA,
        pltpu.SemaphoreType.DMA,
        pltpu.SemaphoreType.DMA,
        pltpu.SemaphoreType.DMA,
        pltpu.SemaphoreType.DMA,
        pltpu.SemaphoreType.DMA,
        pltpu.SemaphoreType.DMA,
        pltpu.SemaphoreType.DMA,
        pltpu.SemaphoreType.DMA,
    ],
)
def _layer(e_hbm, z_hbm, ed_hbm, out_hbm,
           acc, eidx, rows,
           si0, si1, si2, si3, sg0, sg1, sg2, sg3, ss0, ss1, ss2, ss3):
    cid = lax.axis_index("c")
    sid = lax.axis_index("s")
    wid = cid * NS + sid
    si = (si0, si1, si2, si3)
    sg = (sg0, sg1, sg2, sg3)
    ss = (ss0, ss1, ss2, ss3)

    r0 = _al8(sid * ROWS_SC)
    pltpu.sync_copy(z_hbm.at[pl.ds(r0, ROWS_SC)], acc.at[pl.ds(r0, ROWS_SC)])
    plsc.subcore_barrier()

    def fire_idx(b, q):
        pltpu.async_copy(ed_hbm.at[wid * NB + b], eidx.at[q], si[q])

    def wait_idx(q):
        pltpu.make_async_copy(ed_hbm.at[0], eidx.at[q], si[q]).wait()

    def fire_gather(q):
        pltpu.async_copy(e_hbm.at[eidx.at[q, 0]], rows.at[q], sg[q])

    def wait_gather(q):
        pltpu.make_async_copy(e_hbm.at[eidx.at[q, 0]], rows.at[q],
                              sg[q]).wait()

    def fire_scatter(q):
        pltpu.async_copy(rows.at[q], acc.at[eidx.at[q, 1]], ss[q], add=True)

    def wait_scatter(q):
        pltpu.make_async_copy(rows.at[q], acc.at[eidx.at[q, 1]],
                              ss[q]).wait()

    fire_idx(0, 0)
    fire_idx(1, 1)
    wait_idx(0)
    fire_gather(0)
    wait_idx(1)
    fire_gather(1)

    def quarter_iter(bb, r):
        b = bb * 4 + r
        q = r

        wait_gather(q)

        @pl.when(b >= 2)
        def _():
            wait_scatter((r + 2) % 4)

        @pl.when(b + 2 < NB)
        def _():
            fire_idx(b + 2, (r + 2) % 4)

        @pl.loop(0, BLK // LANES)
        def _scale(mi):
            base = pl.multiple_of(mi * LANES, LANES)
            wvec = lax.bitcast_convert_type(
                eidx[q, 2, pl.ds(base, LANES)], jnp.float32)
            dnums = lax.GatherDimensionNumbers(
                offset_dims=(), collapsed_slice_dims=(0,),
                start_index_map=(0,))
            for j in range(LANES):
                idxv = jnp.full((LANES,), j, jnp.int32)
                wb = lax.gather(
                    wvec, idxv[:, None], dnums, (1,),
                    mode=lax.GatherScatterMode.PROMISE_IN_BOUNDS)
                rows[q, base + j] = rows[q, base + j] * wb

        fire_scatter(q)

        @pl.when(b + 2 < NB)
        def _():
            wait_idx((r + 2) % 4)
            fire_gather((r + 2) % 4)

    @pl.loop(0, NB // 4)
    def _bb(bb):
        quarter_iter(bb, 0)
        quarter_iter(bb, 1)
        quarter_iter(bb, 2)
        quarter_iter(bb, 3)

    wait_scatter(2)
    wait_scatter(3)

    plsc.subcore_barrier()
    pltpu.sync_copy(acc.at[pl.ds(r0, ROWS_SC)],
                    out_hbm.at[pl.ds(_al8(cid * NNP + r0), ROWS_SC)])


def _combine_body(last, a_hbm, s_hbm, *refs):
    if last:
        ef_hbm, b0, b1, bs, sem = refs
    else:
        e_hbm, sn_hbm, b0, b1, bs, sem = refs
    wid = lax.axis_index("c") * NS + lax.axis_index("s")

    @pl.loop(0, ROWS_W // CCH)
    def _chunk(k):
        r0 = _al8(wid * ROWS_W + k * CCH)
        d0 = pltpu.async_copy(a_hbm.at[pl.ds(r0, CCH)], b0, sem)
        d1 = pltpu.async_copy(a_hbm.at[pl.ds(_al8(NNP + r0), CCH)], b1, sem)
        d2 = pltpu.async_copy(s_hbm.at[pl.ds(r0, CCH)], bs, sem)
        d0.wait()
        d1.wait()
        d2.wait()

        @pl.loop(0, CCH, unroll=8)
        def _row(r):
            e = b0[r] + b1[r]
            if last:
                b0[r] = (bs[r] + e) * jnp.float32(1.0 / (K_LAYERS + 1))
            else:
                b0[r] = e
                bs[r] = bs[r] + e

        if last:
            pltpu.sync_copy(b0, ef_hbm.at[pl.ds(r0, CCH)])
        else:
            d3 = pltpu.async_copy(b0, e_hbm.at[pl.ds(r0, CCH)], sem)
            d4 = pltpu.async_copy(bs, sn_hbm.at[pl.ds(r0, CCH)], sem)
            d3.wait()
            d4.wait()


_COMBINE_SCRATCH = [
    pltpu.VMEM((CCH, D), jnp.float32),
    pltpu.VMEM((CCH, D), jnp.float32),
    pltpu.VMEM((CCH, D), jnp.float32),
    pltpu.SemaphoreType.DMA,
]

_combine_mid = functools.partial(
    pl.kernel,
    out_type=(jax.ShapeDtypeStruct((NNP, D), jnp.float32),
              jax.ShapeDtypeStruct((NNP, D), jnp.float32)),
    mesh=_MESH,
    compiler_params=_PARAMS,
    scratch_types=_COMBINE_SCRATCH,
)(functools.partial(_combine_body, False))

_combine_last = functools.partial(
    pl.kernel,
    out_type=jax.ShapeDtypeStruct((NNP, D), jnp.float32),
    mesh=_MESH,
    compiler_params=_PARAMS,
    scratch_types=_COMBINE_SCRATCH,
)(functools.partial(_combine_body, True))


N_IDX = 3 * BATCH
G_PER_W = N_IDX // NW


@functools.partial(
    pl.kernel,
    out_type=jax.ShapeDtypeStruct((N_IDX, D), jnp.float32),
    mesh=_MESH,
    compiler_params=_PARAMS,
    scratch_types=[
        pltpu.VMEM((G_PER_W,), jnp.int32),
        pltpu.VMEM((G_PER_W, D), jnp.float32),
        pltpu.SemaphoreType.DMA,
    ],
)
def _batch_gather(e_hbm, idx_hbm, out_hbm, iv, rbuf, sem):
    wid = lax.axis_index("c") * NS + lax.axis_index("s")
    pltpu.sync_copy(idx_hbm.at[pl.ds(_al8(wid * G_PER_W), G_PER_W)], iv)
    descs = [
        pltpu.async_copy(e_hbm.at[iv.at[pl.ds(j * SUB, SUB)]],
                         rbuf.at[pl.ds(j * SUB, SUB)], sem)
        for j in range(G_PER_W // SUB)
    ]
    for d in descs:
        d.wait()
    pltpu.sync_copy(rbuf, out_hbm.at[pl.ds(_al8(wid * G_PER_W), G_PER_W)])


def kernel(users, pos_items, neg_items, user_emb, item_emb,
           edge_src, edge_dst, edge_w):
    E0 = jnp.concatenate([user_emb, item_emb], axis=0)
    E0 = jnp.pad(E0, ((0, NNP - NN), (0, 0)))
    Z = jnp.zeros((NNP, D), jnp.float32)
    pad = NE_PAD - NE
    srcb = jnp.pad(edge_src, (0, pad)).reshape(-1, BLK)
    dstb = jnp.pad(edge_dst, (0, pad)).reshape(-1, BLK)
    wb = jnp.pad(edge_w, (0, pad)).view(jnp.int32).reshape(-1, BLK)
    edata = jnp.stack([srcb, dstb, wb], axis=1)

    E, S = E0, E0
    for k in range(K_LAYERS):
        A = _layer(E, Z, edata)
        if k < K_LAYERS - 1:
            E, S = _combine_mid(A, S)
        else:
            E_final = _combine_last(A, S)

    idx = jnp.concatenate([users, N_USERS + pos_items, N_USERS + neg_items])
    rows = _batch_gather(E_final, idx)
    return (rows[:BATCH], rows[BATCH:2 * BATCH], rows[2 * BATCH:])
